# baseline (speedup 1.0000x reference)
import jax
import jax.numpy as jnp
from jax import lax
from jax.experimental import pallas as pl
from jax.experimental.pallas import tpu as pltpu
from jax.experimental.pallas import tpu_sc as plsc

_H = 224
_W = 224
_P = _H * _W
_B = 8
_C = 192
_R = 2 * _C
_L = 16
_NC, _NS = 2, 16
_NW = _NC * _NS
_WPB = _NW // _B
_ROWS_PER_W = _H // _WPB
_CHUNK = 112
_NGRP = _CHUNK // _L
_CGRP = _C // _L
_SCALE = 2.0 / (_W - 1)


def _floor_i32(x):
    t = x.astype(jnp.int32)
    return t - (t.astype(jnp.float32) > x).astype(jnp.int32)


def _bf16r(x):
    u = lax.bitcast_convert_type(x, jnp.uint32)
    u = (u + jnp.uint32(0x7FFF) + ((u >> jnp.uint32(16)) & jnp.uint32(1)))
    u = u & jnp.uint32(0xFFFF0000)
    return lax.bitcast_convert_type(u, jnp.float32)


def _body(tv_hbm, th_hbm, out_hbm, th_v, i1_v, i2_v,
          wa_v, wb_v, wc_v, wd_v, ob_v, p1_v, p2_v, o_v, sem):
    wid = lax.axis_index("s") * _NC + lax.axis_index("c")
    b = wid // _WPB
    i0 = (wid % _WPB) * _ROWS_PER_W
    boff = b * _P

    pltpu.sync_copy(th_hbm.at[b], th_v)
    tv = _bf16r(th_v[...])
    t00 = tv[0]
    t01 = tv[1]
    t02 = tv[2]
    t10 = tv[3]
    t11 = tv[4]
    t12 = tv[5]
    lane = lax.iota(jnp.int32, _L)

    def chunk_body(u, carry):
        i = i0 + (u >> 1)
        j0 = (u & 1) * _CHUNK
        zf = lane.astype(jnp.float32) * 0.0
        gxv = _bf16r(zf + (i.astype(jnp.float32) * _SCALE - 1.0))

        for g in range(_NGRP):
            j = j0 + g * _L + lane
            gy = _bf16r(j.astype(jnp.float32) * _SCALE - 1.0)
            px = (t00 * gxv + t01 * gy + t02 + 1.0) * (_W * 0.5)
            py = (t10 * gxv + t11 * gy + t12 + 1.0) * (_H * 0.5)
            x0 = _floor_i32(px)
            y0 = _floor_i32(py)
            x1 = jnp.clip(x0 + 1, 0, _W - 1)
            x0 = jnp.clip(x0, 0, _W - 1)
            y1 = jnp.clip(y0 + 1, 0, _H - 1)
            y0 = jnp.clip(y0, 0, _H - 1)
            sl = pl.ds(g * _L, _L)
            i1_v[sl] = y0 * _W + x0 + boff
            i2_v[sl] = y0 * _W + x1 + boff
            ob_v[sl] = (y1 - y0) * _C
            x0f = x0.astype(jnp.float32)
            x1f = x1.astype(jnp.float32)
            y0f = y0.astype(jnp.float32)
            y1f = y1.astype(jnp.float32)
            wa_v[sl] = (x1f - px) * (y1f - py)
            wb_v[sl] = (x1f - px) * (py - y0f)
            wc_v[sl] = (px - x0f) * (y1f - py)
            wd_v[sl] = (px - x0f) * (py - y0f)

        cp1 = pltpu.async_copy(tv_hbm.at[i1_v], p1_v, sem)
        cp2 = pltpu.async_copy(tv_hbm.at[i2_v], p2_v, sem)
        cp1.wait()
        cp2.wait()

        def pt_body(r, c):
            wa = wa_v[pl.ds(r, _L)][0]
            wb = wb_v[pl.ds(r, _L)][0]
            wc = wc_v[pl.ds(r, _L)][0]
            wd = wd_v[pl.ds(r, _L)][0]
            ob = ob_v[pl.ds(r, _L)][0]
            for g2 in range(_CGRP):
                cs = pl.ds(g2 * _L, _L)
                bs = pl.ds(ob + g2 * _L, _L)
                o_v[r, cs] = ((wa * p1_v[r, cs] + wb * p1_v[r, bs])
                              + wc * p2_v[r, cs]) + wd * p2_v[r, bs]
            return c

        lax.fori_loop(0, _CHUNK, pt_body, 0)
        base = boff + i * _W + j0
        pltpu.sync_copy(o_v, out_hbm.at[pl.ds(base, _CHUNK)])
        return carry

    lax.fori_loop(0, _ROWS_PER_W * 2, chunk_body, 0)


_sc_sample = pl.kernel(
    _body,
    out_type=jax.ShapeDtypeStruct((_B * _P, _C), jnp.float32),
    mesh=plsc.VectorSubcoreMesh(
        core_axis_name="c", subcore_axis_name="s",
        num_cores=_NC, num_subcores=_NS),
    compiler_params=pltpu.CompilerParams(
        needs_layout_passes=False, use_tc_tiling_on_sc=True),
    scratch_types=[
        pltpu.VMEM((_L,), jnp.float32),
        pltpu.VMEM((_CHUNK,), jnp.int32),
        pltpu.VMEM((_CHUNK,), jnp.int32),
        pltpu.VMEM((_CHUNK + _L,), jnp.float32),
        pltpu.VMEM((_CHUNK + _L,), jnp.float32),
        pltpu.VMEM((_CHUNK + _L,), jnp.float32),
        pltpu.VMEM((_CHUNK + _L,), jnp.float32),
        pltpu.VMEM((_CHUNK + _L,), jnp.int32),
        pltpu.VMEM((_CHUNK, _R), jnp.float32),
        pltpu.VMEM((_CHUNK, _R), jnp.float32),
        pltpu.VMEM((_CHUNK, _C), jnp.float32),
        pltpu.SemaphoreType.DMA,
    ],
)


def _cat_body(a_ref, b_ref, o_ref):
    o_ref[...] = jnp.concatenate([a_ref[...], b_ref[...]], axis=-1)


_tv_build = pl.pallas_call(
    _cat_body,
    grid=(_B * _H,),
    in_specs=[
        pl.BlockSpec((_W, _C), lambda g: (g, 0)),
        pl.BlockSpec((_W, _C),
                     lambda g: (jnp.where(g % _H == _H - 1, g, g + 1), 0)),
    ],
    out_specs=pl.BlockSpec((_W, _R), lambda g: (g, 0)),
    out_shape=jax.ShapeDtypeStruct((_B * _P, _R), jnp.float32),
)


def kernel(X, affine_transformation):
    x2d = X.reshape(_B * _P, _C)
    tv_table = _tv_build(x2d, x2d)
    th = jnp.zeros((_B, _L), jnp.float32).at[:, :6].set(
        affine_transformation.astype(jnp.float32))
    out = _sc_sample(tv_table, th)
    return out.reshape(_B, _H, _W, _C)


# speedup vs baseline: 1.3964x; 1.3964x over previous
import jax
import jax.numpy as jnp
from jax import lax
from jax.experimental import pallas as pl
from jax.experimental.pallas import tpu as pltpu
from jax.experimental.pallas import tpu_sc as plsc

_H = 224
_W = 224
_P = _H * _W
_B = 8
_C = 192
_R = 2 * _C
_L = 16
_NC, _NS = 2, 16
_NW = _NC * _NS
_WPB = _NW // _B
_ROWS_PER_W = _H // _WPB
_CHUNK = 32
_CPR = _W // _CHUNK
_NCHUNK = _ROWS_PER_W * _CPR
_NGRP = _CHUNK // _L
_CGRP = _C // _L
_SCALE = 2.0 / (_W - 1)


def _floor_i32(x):
    t = x.astype(jnp.int32)
    return t - (t.astype(jnp.float32) > x).astype(jnp.int32)


def _bf16r(x):
    u = lax.bitcast_convert_type(x, jnp.uint32)
    u = (u + jnp.uint32(0x7FFF) + ((u >> jnp.uint32(16)) & jnp.uint32(1)))
    u = u & jnp.uint32(0xFFFF0000)
    return lax.bitcast_convert_type(u, jnp.float32)


def _body(tv_hbm, th_hbm, out_hbm, th_v,
          i1a, i2a, i1b, i2b, waa, wba, wca, wda, oba,
          wab, wbb, wcb, wdb, obb, p1a, p2a, p1b, p2b, o_v, sem):
    wid = lax.axis_index("s") * _NC + lax.axis_index("c")
    b = wid // _WPB
    i0 = (wid % _WPB) * _ROWS_PER_W
    boff = b * _P

    pltpu.sync_copy(th_hbm.at[b], th_v)
    tv = _bf16r(th_v[...])
    t00 = tv[0]
    t01 = tv[1]
    t02 = tv[2]
    t10 = tv[3]
    t11 = tv[4]
    t12 = tv[5]
    lane = lax.iota(jnp.int32, _L)
    zf = lane.astype(jnp.float32) * 0.0

    sets = (
        (i1a, i2a, waa, wba, wca, wda, oba, p1a, p2a),
        (i1b, i2b, wab, wbb, wcb, wdb, obb, p1b, p2b),
    )

    def compute_idx(u, s):
        i1_v, i2_v, wa_v, wb_v, wc_v, wd_v, ob_v = sets[s][:7]
        i = i0 + u // _CPR
        j0 = (u % _CPR) * _CHUNK
        gxv = _bf16r(zf + (i.astype(jnp.float32) * _SCALE - 1.0))
        for g in range(_NGRP):
            j = j0 + g * _L + lane
            gy = _bf16r(j.astype(jnp.float32) * _SCALE - 1.0)
            px = (t00 * gxv + t01 * gy + t02 + 1.0) * (_W * 0.5)
            py = (t10 * gxv + t11 * gy + t12 + 1.0) * (_H * 0.5)
            x0 = _floor_i32(px)
            y0 = _floor_i32(py)
            x1 = jnp.clip(x0 + 1, 0, _W - 1)
            x0 = jnp.clip(x0, 0, _W - 1)
            y1 = jnp.clip(y0 + 1, 0, _H - 1)
            y0 = jnp.clip(y0, 0, _H - 1)
            sl = pl.ds(g * _L, _L)
            i1_v[sl] = y0 * _W + x0 + boff
            i2_v[sl] = y0 * _W + x1 + boff
            ob_v[sl] = (y1 - y0) * _C
            x0f = x0.astype(jnp.float32)
            x1f = x1.astype(jnp.float32)
            y0f = y0.astype(jnp.float32)
            y1f = y1.astype(jnp.float32)
            wa_v[sl] = (x1f - px) * (y1f - py)
            wb_v[sl] = (x1f - px) * (py - y0f)
            wc_v[sl] = (px - x0f) * (y1f - py)
            wd_v[sl] = (px - x0f) * (py - y0f)

    def fire(s):
        i1_v, i2_v = sets[s][:2]
        p1_v, p2_v = sets[s][7:9]
        pltpu.async_copy(tv_hbm.at[i1_v], p1_v, sem)
        pltpu.async_copy(tv_hbm.at[i2_v], p2_v, sem)

    def drain(s):
        p1_v, p2_v = sets[s][7:9]
        pltpu.make_async_copy(tv_hbm.at[pl.ds(0, _CHUNK)], p1_v, sem).wait()
        pltpu.make_async_copy(tv_hbm.at[pl.ds(0, _CHUNK)], p2_v, sem).wait()

    def combine_and_store(u, s):
        wa_v, wb_v, wc_v, wd_v, ob_v = sets[s][2:7]
        p1_v, p2_v = sets[s][7:9]

        def pt_body(r, c):
            wa = wa_v[pl.ds(r, _L)][0]
            wb = wb_v[pl.ds(r, _L)][0]
            wc = wc_v[pl.ds(r, _L)][0]
            wd = wd_v[pl.ds(r, _L)][0]
            ob = ob_v[pl.ds(r, _L)][0]
            for g2 in range(_CGRP):
                cs = pl.ds(g2 * _L, _L)
                bs = pl.ds(ob + g2 * _L, _L)
                o_v[r, cs] = ((wa * p1_v[r, cs] + wb * p1_v[r, bs])
                              + wc * p2_v[r, cs]) + wd * p2_v[r, bs]
            return c

        lax.fori_loop(0, _CHUNK, pt_body, 0)
        i = i0 + u // _CPR
        j0 = (u % _CPR) * _CHUNK
        base = boff + i * _W + j0
        pltpu.sync_copy(o_v, out_hbm.at[pl.ds(base, _CHUNK)])

    compute_idx(0, 0)
    fire(0)

    def pair_body(t, carry):
        u0 = 2 * t
        compute_idx(u0 + 1, 1)
        fire(1)
        drain(0)
        combine_and_store(u0, 0)

        @pl.when(t < _NCHUNK // 2 - 1)
        def _():
            compute_idx(u0 + 2, 0)
            fire(0)

        drain(1)
        combine_and_store(u0 + 1, 1)
        return carry

    lax.fori_loop(0, _NCHUNK // 2, pair_body, 0)


_sc_sample = pl.kernel(
    _body,
    out_type=jax.ShapeDtypeStruct((_B * _P, _C), jnp.float32),
    mesh=plsc.VectorSubcoreMesh(
        core_axis_name="c", subcore_axis_name="s",
        num_cores=_NC, num_subcores=_NS),
    compiler_params=pltpu.CompilerParams(
        needs_layout_passes=False, use_tc_tiling_on_sc=True),
    scratch_types=[
        pltpu.VMEM((_L,), jnp.float32),
        pltpu.VMEM((_CHUNK,), jnp.int32),
        pltpu.VMEM((_CHUNK,), jnp.int32),
        pltpu.VMEM((_CHUNK,), jnp.int32),
        pltpu.VMEM((_CHUNK,), jnp.int32),
        pltpu.VMEM((_CHUNK + _L,), jnp.float32),
        pltpu.VMEM((_CHUNK + _L,), jnp.float32),
        pltpu.VMEM((_CHUNK + _L,), jnp.float32),
        pltpu.VMEM((_CHUNK + _L,), jnp.float32),
        pltpu.VMEM((_CHUNK + _L,), jnp.int32),
        pltpu.VMEM((_CHUNK + _L,), jnp.float32),
        pltpu.VMEM((_CHUNK + _L,), jnp.float32),
        pltpu.VMEM((_CHUNK + _L,), jnp.float32),
        pltpu.VMEM((_CHUNK + _L,), jnp.float32),
        pltpu.VMEM((_CHUNK + _L,), jnp.int32),
        pltpu.VMEM((_CHUNK, _R), jnp.float32),
        pltpu.VMEM((_CHUNK, _R), jnp.float32),
        pltpu.VMEM((_CHUNK, _R), jnp.float32),
        pltpu.VMEM((_CHUNK, _R), jnp.float32),
        pltpu.VMEM((_CHUNK, _C), jnp.float32),
        pltpu.SemaphoreType.DMA,
    ],
)


def kernel(X, affine_transformation):
    xs = jnp.concatenate([X[:, 1:], X[:, -1:]], axis=1)
    tv_table = jnp.concatenate([X, xs], axis=-1).reshape(_B * _P, _R)
    th = jnp.zeros((_B, _L), jnp.float32).at[:, :6].set(
        affine_transformation.astype(jnp.float32))
    out = _sc_sample(tv_table, th)
    return out.reshape(_B, _H, _W, _C)
